# baseline (speedup 1.0000x reference)
import functools

import jax
import jax.numpy as jnp
from jax import lax
from jax.experimental import pallas as pl
from jax.experimental.pallas import tpu as pltpu
from jax.experimental.pallas import tpu_sc as plsc
from jax.experimental.layout import Format, Layout, with_layout_constraint

B = 16384
R = 32
NC = 2
NS = 16
L = 16
NW = NC * NS
BPW = B // NW
GRP = BPW // L


def _lane_take(x, idx):
    dnums = lax.GatherDimensionNumbers(
        offset_dims=(), collapsed_slice_dims=(0,), start_index_map=(0,))
    return lax.gather(x, idx[:, None], dnums, (1,),
                      mode=lax.GatherScatterMode.PROMISE_IN_BOUNDS)


def _body(u_hbm, i_hbm, j_hbm, U_hbm, V_hbm, out_hbm,
          idx_u, idx_i, idx_j, rows_u, rows_i, rows_j, out_v, sem):
    wid = lax.axis_index("s") * NC + lax.axis_index("c")
    base = wid * BPW

    pltpu.sync_copy(u_hbm.at[pl.ds(base, BPW)], idx_u)
    pltpu.sync_copy(i_hbm.at[pl.ds(base, BPW)], idx_i)
    pltpu.sync_copy(j_hbm.at[pl.ds(base, BPW)], idx_j)

    def fire(g, carry):
        d = pl.ds(g * L, L)
        pltpu.async_copy(U_hbm.at[idx_u[d]], rows_u.at[d], sem)
        pltpu.async_copy(V_hbm.at[idx_i[d]], rows_i.at[d], sem)
        pltpu.async_copy(V_hbm.at[idx_j[d]], rows_j.at[d], sem)
        return carry

    lax.fori_loop(0, GRP, fire, 0)
    pltpu.make_async_copy(U_hbm.at[pl.ds(0, BPW)], rows_u, sem).wait()
    pltpu.make_async_copy(V_hbm.at[pl.ds(0, BPW)], rows_i, sem).wait()
    pltpu.make_async_copy(V_hbm.at[pl.ds(0, BPW)], rows_j, sem).wait()

    lane = lax.iota(jnp.int32, L)
    rots = [(lane + off) & (L - 1) for off in (8, 4, 2, 1)]
    zero = jnp.zeros((L,), jnp.float32)

    def group(g, carry):
        gb = g * L
        acc = zero
        for t in range(L):
            r = gb + t
            u0 = rows_u[r, pl.ds(0, L)]
            u1 = rows_u[r, pl.ds(L, L)]
            i0 = rows_i[r, pl.ds(0, L)]
            i1 = rows_i[r, pl.ds(L, L)]
            j0 = rows_j[r, pl.ds(0, L)]
            j1 = rows_j[r, pl.ds(L, L)]
            s = u0 * (i0 - j0) + u1 * (i1 - j1)
            for rot in rots:
                s = s + _lane_take(s, rot)
            acc = jnp.where(lane == t, s, acc)
        out_v[pl.ds(gb, L)] = 1.0 / (1.0 + jnp.exp(-acc))
        return carry

    lax.fori_loop(0, GRP, group, 0)

    pltpu.sync_copy(out_v, out_hbm.at[pl.ds(base, BPW)])


@jax.jit
def _run(u, i, j, U, V):
    mesh = plsc.VectorSubcoreMesh(core_axis_name="c", subcore_axis_name="s")
    f = functools.partial(
        pl.kernel,
        mesh=mesh,
        out_type=jax.ShapeDtypeStruct((B,), jnp.float32),
        scratch_types=[
            pltpu.VMEM((BPW,), jnp.int32),
            pltpu.VMEM((BPW,), jnp.int32),
            pltpu.VMEM((BPW,), jnp.int32),
            pltpu.VMEM((BPW, R), jnp.float32),
            pltpu.VMEM((BPW, R), jnp.float32),
            pltpu.VMEM((BPW, R), jnp.float32),
            pltpu.VMEM((BPW,), jnp.float32),
            pltpu.SemaphoreType.DMA,
        ],
        compiler_params=pltpu.CompilerParams(use_tc_tiling_on_sc=False),
    )(_body)
    return f(u, i, j, U, V)


_SC_LAYOUT = Layout(major_to_minor=(0, 1), tiling=((8,),))


def kernel(u, i, j, U, V):
    U2 = with_layout_constraint(U, _SC_LAYOUT)
    V2 = with_layout_constraint(V, _SC_LAYOUT)
    return _run(u.astype(jnp.int32), i.astype(jnp.int32), j.astype(jnp.int32),
                U2, V2)
